# Initial kernel scaffold; baseline (speedup 1.0000x reference)
import jax
import jax.numpy as jnp
from jax.experimental import pallas as pl


def kernel(code_holiday, code_weather, code_weather_detail, code_month, code_dayofweek, code_hour, W_holiday, W_weather, W_weather_detail, W_month, W_dayofweek, W_hour):
    raise NotImplementedError("write your pallas kernel here")


# speedup vs baseline: 6.6115x; 6.6115x over previous
import functools

import jax
import jax.numpy as jnp
from jax import lax
from jax.experimental import pallas as pl
from jax.experimental.pallas import tpu as pltpu
from jax.experimental.pallas import tpu_sc as plsc

B = 16384
D = 64
NC = 2
NS = 16
NW = NC * NS
BPW = B // NW
CHUNK = 128
NCHUNK = BPW // CHUNK
LANES = 16

_MESH = plsc.VectorSubcoreMesh(core_axis_name="c", subcore_axis_name="s")


@functools.partial(
    pl.kernel,
    mesh=_MESH,
    out_type=jax.ShapeDtypeStruct((B, 6 * D), jnp.float32),
    scratch_types=[
        pltpu.VMEM((6, BPW), jnp.int32),
        pltpu.VMEM((NCHUNK, CHUNK), jnp.int32),
        pltpu.VMEM((NCHUNK, CHUNK), jnp.int32),
        pltpu.VMEM((NCHUNK, CHUNK), jnp.int32),
        pltpu.VMEM((CHUNK, 2 * D), jnp.float32),
        pltpu.VMEM((CHUNK, 2 * D), jnp.float32),
        pltpu.VMEM((CHUNK, 2 * D), jnp.float32),
        pltpu.SemaphoreType.DMA,
    ],
)
def _sc_embed(t12, t34, t56, c1, c2, c3, c4, c5, c6, out,
              codes, idx12, idx34, idx56, b12, b34, b56, sem):
    wid = lax.axis_index("s") * NC + lax.axis_index("c")
    base = wid * BPW

    pltpu.sync_copy(c1.at[pl.ds(base, BPW)], codes.at[0])
    pltpu.sync_copy(c2.at[pl.ds(base, BPW)], codes.at[1])
    pltpu.sync_copy(c3.at[pl.ds(base, BPW)], codes.at[2])
    pltpu.sync_copy(c4.at[pl.ds(base, BPW)], codes.at[3])
    pltpu.sync_copy(c5.at[pl.ds(base, BPW)], codes.at[4])
    pltpu.sync_copy(c6.at[pl.ds(base, BPW)], codes.at[5])

    for c in range(NCHUNK):
        for k in range(CHUNK // LANES):
            s = c * CHUNK + k * LANES
            sl = pl.ds(s, LANES)
            ksl = pl.ds(k * LANES, LANES)
            idx12[c, ksl] = codes[0, sl] * 11 + codes[1, sl]
            idx34[c, ksl] = codes[2, sl] * 12 + codes[3, sl]
            idx56[c, ksl] = codes[4, sl] * 24 + codes[5, sl]

    for c in range(NCHUNK):
        cp1 = pltpu.async_copy(t12.at[idx12.at[c]], b12, sem)
        cp2 = pltpu.async_copy(t34.at[idx34.at[c]], b34, sem)
        cp3 = pltpu.async_copy(t56.at[idx56.at[c]], b56, sem)
        cp1.wait()
        cp2.wait()
        cp3.wait()
        r0 = base + c * CHUNK
        pltpu.sync_copy(b12, out.at[pl.ds(r0, CHUNK), pl.ds(0, 2 * D)])
        pltpu.sync_copy(b34, out.at[pl.ds(r0, CHUNK), pl.ds(2 * D, 2 * D)])
        pltpu.sync_copy(b56, out.at[pl.ds(r0, CHUNK), pl.ds(4 * D, 2 * D)])


def kernel(code_holiday, code_weather, code_weather_detail, code_month,
           code_dayofweek, code_hour, W_holiday, W_weather, W_weather_detail,
           W_month, W_dayofweek, W_hour):
    t12 = jnp.concatenate([
        jnp.broadcast_to(W_holiday[:, None, :], (12, 11, D)),
        jnp.broadcast_to(W_weather[None, :, :], (12, 11, D)),
    ], axis=2).reshape(12 * 11, 2 * D)
    t34 = jnp.concatenate([
        jnp.broadcast_to(W_weather_detail[:, None, :], (38, 12, D)),
        jnp.broadcast_to(W_month[None, :, :], (38, 12, D)),
    ], axis=2).reshape(38 * 12, 2 * D)
    t56 = jnp.concatenate([
        jnp.broadcast_to(W_dayofweek[:, None, :], (7, 24, D)),
        jnp.broadcast_to(W_hour[None, :, :], (7, 24, D)),
    ], axis=2).reshape(7 * 24, 2 * D)

    codes = [c.astype(jnp.int32) for c in (
        code_holiday, code_weather, code_weather_detail,
        code_month, code_dayofweek, code_hour)]
    return _sc_embed(t12, t34, t56, *codes)
